# baseline (speedup 1.0000x reference)
import functools

import jax
import jax.numpy as jnp
from jax import lax
from jax.experimental import pallas as pl
from jax.experimental.pallas import tpu as pltpu
from jax.experimental.pallas import tpu_sc as plsc

NC = 2
NS = 16
NW = NC * NS
L = 16


def _pad_flat(W):
    V, D = W.shape
    W3 = W.reshape(V // 8, 8, D)
    W3 = jnp.pad(W3, ((0, 0), (0, 0), (0, 128 - D)))
    return W3.reshape(-1)


def kernel(user, item, W_user, W_item):
    B = user.shape[0]
    D = W_user.shape[1]
    b_per_w = B // NW
    n_streams = (b_per_w * D) // 128

    mesh = plsc.VectorSubcoreMesh(core_axis_name="c", subcore_axis_name="s")

    @functools.partial(
        pl.kernel,
        out_type=[
            jax.ShapeDtypeStruct((B * D,), jnp.float32),
            jax.ShapeDtypeStruct((B * D,), jnp.float32),
        ],
        mesh=mesh,
        compiler_params=pltpu.CompilerParams(
            use_tc_tiling_on_sc=False, needs_layout_passes=False),
        scratch_types=[
            pltpu.VMEM((b_per_w,), jnp.int32),
            pltpu.VMEM((b_per_w,), jnp.int32),
            pltpu.VMEM((b_per_w * D,), jnp.int32),
            pltpu.VMEM((b_per_w * D,), jnp.int32),
            pltpu.VMEM((b_per_w * D,), jnp.float32),
            pltpu.VMEM((b_per_w * D,), jnp.float32),
            pltpu.SemaphoreType.DMA,
            pltpu.SemaphoreType.DMA,
        ],
    )
    def body(user_hbm, item_hbm, wu_hbm, wi_hbm, out_u, out_i,
             uidx_v, iidx_v, uexp_v, iexp_v, urow_v, irow_v, sem_u, sem_i):
        wid = lax.axis_index("s") * NC + lax.axis_index("c")
        base = wid * b_per_w
        pltpu.sync_copy(user_hbm.at[pl.ds(base, b_per_w)], uidx_v)
        pltpu.sync_copy(item_hbm.at[pl.ds(base, b_per_w)], iidx_v)

        iota = lax.iota(jnp.int32, L)
        for g in range(b_per_w // L):
            pos = (g * L + iota) * D
            ui = uidx_v[pl.ds(g * L, L)]
            ii = iidx_v[pl.ds(g * L, L)]
            ubase = lax.shift_right_logical(ui, 3) * 1024 + (ui & 7) * 128
            ibase = lax.shift_right_logical(ii, 3) * 1024 + (ii & 7) * 128
            for d in range(D):
                plsc.store_scatter(uexp_v, [pos + d], ubase + d)
                plsc.store_scatter(iexp_v, [pos + d], ibase + d)

        waits = []
        for s in range(n_streams):
            sl = pl.ds(s * 128, 128)
            waits.append(pltpu.async_copy(
                wu_hbm.at[uexp_v.at[sl]], urow_v.at[sl], sem_u))
            waits.append(pltpu.async_copy(
                wi_hbm.at[iexp_v.at[sl]], irow_v.at[sl], sem_i))
        for w in waits:
            w.wait()
        pltpu.sync_copy(urow_v, out_u.at[pl.ds(base * D, b_per_w * D)])
        pltpu.sync_copy(irow_v, out_i.at[pl.ds(base * D, b_per_w * D)])

    out_u, out_i = body(user, item, _pad_flat(W_user), _pad_flat(W_item))
    return (out_u.reshape(B, D), out_i.reshape(B, D))


# speedup vs baseline: 1.0965x; 1.0965x over previous
import functools

import jax
import jax.numpy as jnp
from jax import lax
from jax.experimental import pallas as pl
from jax.experimental.pallas import tpu as pltpu
from jax.experimental.pallas import tpu_sc as plsc

NC = 2
NS = 16
NW = NC * NS


def kernel(user, item, W_user, W_item):
    B = user.shape[0]
    D = W_user.shape[1]
    assert B % NW == 0
    b_per_w = B // NW

    mesh = plsc.VectorSubcoreMesh(core_axis_name="c", subcore_axis_name="s")

    @functools.partial(
        pl.kernel,
        out_type=[
            jax.ShapeDtypeStruct((B, D), jnp.float32),
            jax.ShapeDtypeStruct((B, D), jnp.float32),
        ],
        mesh=mesh,
        compiler_params=pltpu.CompilerParams(use_tc_tiling_on_sc=False),
        scratch_types=[
            pltpu.VMEM((b_per_w,), jnp.int32),
            pltpu.VMEM((b_per_w, D), jnp.float32),
            pltpu.VMEM((b_per_w,), jnp.int32),
            pltpu.VMEM((b_per_w, D), jnp.float32),
            pltpu.SemaphoreType.DMA,
            pltpu.SemaphoreType.DMA,
        ],
    )
    def body(user_hbm, item_hbm, wu_hbm, wi_hbm, out_u, out_i,
             uidx_v, urow_v, iidx_v, irow_v, sem_u, sem_i):
        wid = lax.axis_index("s") * NC + lax.axis_index("c")
        base = wid * b_per_w
        pltpu.sync_copy(user_hbm.at[pl.ds(base, b_per_w)], uidx_v)
        pltpu.sync_copy(item_hbm.at[pl.ds(base, b_per_w)], iidx_v)
        cu = pltpu.async_copy(wu_hbm.at[uidx_v], urow_v, sem_u)
        ci = pltpu.async_copy(wi_hbm.at[iidx_v], irow_v, sem_i)
        cu.wait()
        ci.wait()
        pltpu.sync_copy(urow_v, out_u.at[pl.ds(base, b_per_w)])
        pltpu.sync_copy(irow_v, out_i.at[pl.ds(base, b_per_w)])

    return tuple(body(user, item, W_user, W_item))


# speedup vs baseline: 1.4382x; 1.3116x over previous
import functools

import jax
import jax.numpy as jnp
from jax import lax
from jax.experimental import pallas as pl
from jax.experimental.pallas import tpu as pltpu
from jax.experimental.pallas import tpu_sc as plsc

NC = 2
NS = 16
NW = NC * NS
CH = 16


def kernel(user, item, W_user, W_item):
    B = user.shape[0]
    D = W_user.shape[1]
    b_per_w = B // NW
    n_chunks = b_per_w // CH

    mesh = plsc.VectorSubcoreMesh(core_axis_name="c", subcore_axis_name="s")

    @functools.partial(
        pl.kernel,
        out_type=[
            jax.ShapeDtypeStruct((B * D,), jnp.float32),
            jax.ShapeDtypeStruct((B * D,), jnp.float32),
        ],
        mesh=mesh,
        scratch_types=[
            pltpu.VMEM((b_per_w,), jnp.int32),
            pltpu.VMEM((b_per_w,), jnp.int32),
            pltpu.VMEM((2, CH * 8, D), jnp.float32),
            pltpu.VMEM((2, CH * 8, D), jnp.float32),
            pltpu.VMEM((b_per_w * D,), jnp.float32),
            pltpu.VMEM((b_per_w * D,), jnp.float32),
            pltpu.SemaphoreType.DMA,
            pltpu.SemaphoreType.DMA,
            pltpu.SemaphoreType.DMA,
            pltpu.SemaphoreType.DMA,
        ],
    )
    def body(user_hbm, item_hbm, wu_hbm, wi_hbm, out_u, out_i,
             uidx_v, iidx_v, ublk_v, iblk_v, urow_v, irow_v,
             sem_u0, sem_u1, sem_i0, sem_i1):
        sems_u = (sem_u0, sem_u1)
        sems_i = (sem_i0, sem_i1)
        wid = lax.axis_index("s") * NC + lax.axis_index("c")
        base = wid * b_per_w
        pltpu.sync_copy(user_hbm.at[pl.ds(base, b_per_w)], uidx_v)
        pltpu.sync_copy(item_hbm.at[pl.ds(base, b_per_w)], iidx_v)

        def blk_starts(vec):
            return lax.shift_right_logical(vec, 3) * 8

        def fire_chunk(c, buf):
            ub = blk_starts(uidx_v[pl.ds(c * CH, CH)])
            ib = blk_starts(iidx_v[pl.ds(c * CH, CH)])
            for k in range(CH):
                u0 = pl.multiple_of(ub[k], 8)
                i0 = pl.multiple_of(ib[k], 8)
                slot = k * 8
                pltpu.async_copy(
                    wu_hbm.at[pl.ds(u0, 8), :],
                    ublk_v.at[buf].at[pl.ds(slot, 8), :], sems_u[buf])
                pltpu.async_copy(
                    wi_hbm.at[pl.ds(i0, 8), :],
                    iblk_v.at[buf].at[pl.ds(slot, 8), :], sems_i[buf])

        def drain(buf):
            pltpu.make_async_copy(
                wu_hbm.at[pl.ds(0, CH * 8), :], ublk_v.at[0],
                sems_u[buf]).wait()
            pltpu.make_async_copy(
                wi_hbm.at[pl.ds(0, CH * 8), :], iblk_v.at[0],
                sems_i[buf]).wait()

        iota8 = lax.iota(jnp.int32, CH) * 8

        def extract_chunk(c, buf):
            us = (uidx_v[pl.ds(c * CH, CH)] & 7) + iota8
            isv = (iidx_v[pl.ds(c * CH, CH)] & 7) + iota8
            for k in range(CH):
                j = c * CH + k
                urow_v[pl.ds(j * D, D)] = ublk_v[buf, us[k], :]
                irow_v[pl.ds(j * D, D)] = iblk_v[buf, isv[k], :]

        fire_chunk(0, 0)
        for c in range(1, n_chunks):
            fire_chunk(c, c % 2)
            drain((c - 1) % 2)
            extract_chunk(c - 1, (c - 1) % 2)
        drain((n_chunks - 1) % 2)
        extract_chunk(n_chunks - 1, (n_chunks - 1) % 2)

        pltpu.sync_copy(urow_v, out_u.at[pl.ds(base * D, b_per_w * D)])
        pltpu.sync_copy(irow_v, out_i.at[pl.ds(base * D, b_per_w * D)])

    out_u, out_i = body(user, item, W_user, W_item)
    return (out_u.reshape(B, D), out_i.reshape(B, D))


# speedup vs baseline: 1.4585x; 1.0141x over previous
import functools

import jax
import jax.numpy as jnp
from jax import lax
from jax.experimental import pallas as pl
from jax.experimental.pallas import tpu as pltpu
from jax.experimental.pallas import tpu_sc as plsc

NC = 2
NS = 16
NW = NC * NS
CH = 16


def kernel(user, item, W_user, W_item):
    B = user.shape[0]
    D = W_user.shape[1]
    b_per_w = B // NW
    n_chunks = b_per_w // CH

    mesh = plsc.VectorSubcoreMesh(core_axis_name="c", subcore_axis_name="s")

    @functools.partial(
        pl.kernel,
        out_type=[
            jax.ShapeDtypeStruct((B * D,), jnp.float32),
            jax.ShapeDtypeStruct((B * D,), jnp.float32),
        ],
        mesh=mesh,
        scratch_types=[
            pltpu.VMEM((b_per_w,), jnp.int32),
            pltpu.VMEM((b_per_w,), jnp.int32),
            pltpu.VMEM((3, CH * 8, D), jnp.float32),
            pltpu.VMEM((3, CH * 8, D), jnp.float32),
            pltpu.VMEM((b_per_w * D,), jnp.float32),
            pltpu.VMEM((b_per_w * D,), jnp.float32),
            pltpu.SemaphoreType.DMA,
            pltpu.SemaphoreType.DMA,
            pltpu.SemaphoreType.DMA,
            pltpu.SemaphoreType.DMA,
            pltpu.SemaphoreType.DMA,
            pltpu.SemaphoreType.DMA,
        ],
    )
    def body(user_hbm, item_hbm, wu_hbm, wi_hbm, out_u, out_i,
             uidx_v, iidx_v, ublk_v, iblk_v, urow_v, irow_v,
             sem_u0, sem_u1, sem_u2, sem_i0, sem_i1, sem_i2):
        sems_u = (sem_u0, sem_u1, sem_u2)
        sems_i = (sem_i0, sem_i1, sem_i2)
        wid = lax.axis_index("s") * NC + lax.axis_index("c")
        base = wid * b_per_w
        pltpu.sync_copy(user_hbm.at[pl.ds(base, b_per_w)], uidx_v)
        pltpu.sync_copy(item_hbm.at[pl.ds(base, b_per_w)], iidx_v)

        def blk_starts(vec):
            return lax.shift_right_logical(vec, 3) * 8

        def fire_chunk(c, buf):
            ub = blk_starts(uidx_v[pl.ds(c * CH, CH)])
            ib = blk_starts(iidx_v[pl.ds(c * CH, CH)])
            for k in range(CH):
                u0 = pl.multiple_of(ub[k], 8)
                i0 = pl.multiple_of(ib[k], 8)
                slot = k * 8
                pltpu.async_copy(
                    wu_hbm.at[pl.ds(u0, 8), :],
                    ublk_v.at[buf].at[pl.ds(slot, 8), :], sems_u[buf])
                pltpu.async_copy(
                    wi_hbm.at[pl.ds(i0, 8), :],
                    iblk_v.at[buf].at[pl.ds(slot, 8), :], sems_i[buf])

        def drain(buf):
            pltpu.make_async_copy(
                wu_hbm.at[pl.ds(0, CH * 8), :], ublk_v.at[0],
                sems_u[buf]).wait()
            pltpu.make_async_copy(
                wi_hbm.at[pl.ds(0, CH * 8), :], iblk_v.at[0],
                sems_i[buf]).wait()

        iota8 = lax.iota(jnp.int32, CH) * 8

        def extract_chunk(c, buf):
            us = (uidx_v[pl.ds(c * CH, CH)] & 7) + iota8
            isv = (iidx_v[pl.ds(c * CH, CH)] & 7) + iota8
            for k in range(CH):
                j = c * CH + k
                urow_v[pl.ds(j * D, D)] = ublk_v[buf, us[k], :]
                irow_v[pl.ds(j * D, D)] = iblk_v[buf, isv[k], :]

        fire_chunk(0, 0)
        fire_chunk(1, 1)
        for c in range(2, n_chunks):
            fire_chunk(c, c % 3)
            drain((c - 2) % 3)
            extract_chunk(c - 2, (c - 2) % 3)
        for c in range(n_chunks - 2, n_chunks):
            drain(c % 3)
            extract_chunk(c, c % 3)

        pltpu.sync_copy(urow_v, out_u.at[pl.ds(base * D, b_per_w * D)])
        pltpu.sync_copy(irow_v, out_i.at[pl.ds(base * D, b_per_w * D)])

    out_u, out_i = body(user, item, W_user, W_item)
    return (out_u.reshape(B, D), out_i.reshape(B, D))
